# baseline (speedup 1.0000x reference)
import jax
import jax.numpy as jnp
from jax.experimental import pallas as pl
from jax.experimental.pallas import tpu as pltpu


def _scatter_body(idx_ref, mem_ref, val_ref, out_ref):
    del idx_ref, mem_ref
    out_ref[...] = val_ref[...]


def kernel(mem, idx, val):
    M, D = mem.shape
    B = val.shape[0]
    grid_spec = pltpu.PrefetchScalarGridSpec(
        num_scalar_prefetch=1,
        grid=(B,),
        in_specs=[
            pl.BlockSpec(memory_space=pltpu.HBM),
            pl.BlockSpec((1, 1, D), lambda b, idx_ref: (b, 0, 0)),
        ],
        out_specs=pl.BlockSpec((1, 1, D), lambda b, idx_ref: (idx_ref[b], 0, 0)),
    )
    out = pl.pallas_call(
        _scatter_body,
        grid_spec=grid_spec,
        out_shape=jax.ShapeDtypeStruct((M, 1, D), mem.dtype),
        input_output_aliases={1: 0},
    )(idx, mem.reshape(M, 1, D), val.reshape(B, 1, D))
    return out.reshape(M, D)
